# baseline (speedup 1.0000x reference)
import functools

import jax
import jax.numpy as jnp
from jax import lax
from jax.experimental import pallas as pl
from jax.experimental.pallas import tpu as pltpu
from jax.experimental.pallas import tpu_sc as plsc

VOCAB = 100000
B = 32
T = 2048
D = 128
C = 128
NB = 16
NBUF = 6
LOOKAHEAD = 5
LANES = 16


def _emb_body(idx_hbm, wtr_hbm, wpe_hbm, out_hbm,
              idx_v, wpe_v, bufs, sems, sem_i, sem_p):
    h = lax.axis_index("c")
    tc = lax.axis_index("s")

    t0 = tc * C
    b0 = h * NB

    sem_g = sems[:2 * NBUF]
    sem_w = sems[2 * NBUF:]

    idx_cp = pltpu.async_copy(
        idx_hbm.at[pl.ds(b0, NB), pl.ds(t0, C)], idx_v, sem_i)
    wpe_cp = pltpu.async_copy(wpe_hbm.at[pl.ds(t0, C)], wpe_v, sem_p)
    idx_cp.wait()

    H = C // 2

    def start_gather_half(j, half):
        s = j % NBUF
        return pltpu.async_copy(
            wtr_hbm.at[idx_v.at[j, pl.ds(half * H, H)]],
            bufs.at[s, pl.ds(half * H, H)],
            sem_g[2 * s + half])

    gd = [[None, None] for _ in range(NB)]
    wd = [[None, None] for _ in range(NB)]

    for j in range(LOOKAHEAD):
        gd[j][0] = start_gather_half(j, 0)
        gd[j][1] = start_gather_half(j, 1)
    wpe_cp.wait()

    for j in range(NB):
        s = j % NBUF
        for half in range(2):
            gd[j][half].wait()

            @pl.loop(0, H)
            def _per_row(r, s=s, half=half):
                ro = r + half * H
                for k in range(D // LANES):
                    sl = pl.ds(k * LANES, LANES)
                    plsc.addupdate(bufs.at[s, ro, sl], wpe_v[ro, sl])

            wd[j][half] = pltpu.async_copy(
                bufs.at[s, pl.ds(half * H, H)],
                out_hbm.at[b0 + j, pl.ds(t0 + half * H, H)],
                sem_w[2 * s + half])

        nj = j + LOOKAHEAD
        if nj < NB:
            pj = nj - NBUF
            if pj >= 0:
                wd[pj][0].wait()
                wd[pj][1].wait()
            gd[nj][0] = start_gather_half(nj, 0)
            gd[nj][1] = start_gather_half(nj, 1)

    for j in range(NB - NBUF, NB):
        wd[j][0].wait()
        wd[j][1].wait()


@functools.partial(
    pl.kernel,
    out_type=jax.ShapeDtypeStruct((B, T, D), jnp.float32),
    mesh=plsc.VectorSubcoreMesh(core_axis_name="c", subcore_axis_name="s"),
    scratch_types=[
        pltpu.VMEM((NB, C), jnp.int32),
        pltpu.VMEM((C, D), jnp.float32),
        pltpu.VMEM((NBUF, C, D), jnp.float32),
        [pltpu.SemaphoreType.DMA] * (4 * NBUF),
        pltpu.SemaphoreType.DMA,
        pltpu.SemaphoreType.DMA,
    ],
)
def _emb_kernel(idx_hbm, wtr_hbm, wpe_hbm, out_hbm, idx_v, wpe_v, bufs, sems,
                sem_i, sem_p):
    _emb_body(idx_hbm, wtr_hbm, wpe_hbm, out_hbm, idx_v, wpe_v, bufs, sems,
              sem_i, sem_p)


def kernel(idx, wtr, wpe):
    idx = idx.astype(jnp.int32)
    return _emb_kernel(idx, wtr, wpe)


# speedup vs baseline: 1.0139x; 1.0139x over previous
import functools

import jax
import jax.numpy as jnp
from jax import lax
from jax.experimental import pallas as pl
from jax.experimental.pallas import tpu as pltpu
from jax.experimental.pallas import tpu_sc as plsc

VOCAB = 100000
B = 32
T = 2048
D = 128
C = 128
NB = 16
NBUF = 6
LOOKAHEAD = 5
LANES = 16


def _emb_body(idx_hbm, wtr_hbm, wpe_hbm, out_hbm,
              idx_v, wpe_v, bufs, sems, sem_i, sem_p):
    h = lax.axis_index("c")
    tc = lax.axis_index("s")

    t0 = tc * C
    b0 = h * NB

    sem_g = sems[:NBUF]
    sem_w = sems[NBUF:]

    idx_cp = pltpu.async_copy(
        idx_hbm.at[pl.ds(b0, NB), pl.ds(t0, C)], idx_v, sem_i)
    wpe_cp = pltpu.async_copy(wpe_hbm.at[pl.ds(t0, C)], wpe_v, sem_p)
    idx_cp.wait()

    def start_gather(j):
        s = j % NBUF
        return pltpu.async_copy(wtr_hbm.at[idx_v.at[j]], bufs.at[s], sem_g[s])

    gd = [None] * NB
    wd = [None] * NB

    for j in range(LOOKAHEAD):
        gd[j] = start_gather(j)
    wpe_cp.wait()

    for j in range(NB):
        s = j % NBUF
        gd[j].wait()

        @pl.loop(0, C, unroll=2)
        def _per_row(r, s=s):
            for k in range(D // LANES):
                sl = pl.ds(k * LANES, LANES)
                plsc.addupdate(bufs.at[s, r, sl], wpe_v[r, sl])

        wd[j] = pltpu.async_copy(
            bufs.at[s], out_hbm.at[b0 + j, pl.ds(t0, C)], sem_w[s])

        nj = j + LOOKAHEAD
        if nj < NB:
            pj = nj - NBUF
            if pj >= 0:
                wd[pj].wait()
            gd[nj] = start_gather(nj)

    for j in range(NB - NBUF, NB):
        if wd[j] is not None and j >= 0:
            wd[j].wait()


@functools.partial(
    pl.kernel,
    out_type=jax.ShapeDtypeStruct((B, T, D), jnp.float32),
    mesh=plsc.VectorSubcoreMesh(core_axis_name="c", subcore_axis_name="s"),
    scratch_types=[
        pltpu.VMEM((NB, C), jnp.int32),
        pltpu.VMEM((C, D), jnp.float32),
        pltpu.VMEM((NBUF, C, D), jnp.float32),
        [pltpu.SemaphoreType.DMA] * (2 * NBUF),
        pltpu.SemaphoreType.DMA,
        pltpu.SemaphoreType.DMA,
    ],
)
def _emb_kernel(idx_hbm, wtr_hbm, wpe_hbm, out_hbm, idx_v, wpe_v, bufs, sems,
                sem_i, sem_p):
    _emb_body(idx_hbm, wtr_hbm, wpe_hbm, out_hbm, idx_v, wpe_v, bufs, sems,
              sem_i, sem_p)


def kernel(idx, wtr, wpe):
    idx = idx.astype(jnp.int32)
    return _emb_kernel(idx, wtr, wpe)


# speedup vs baseline: 1.0406x; 1.0263x over previous
import functools

import jax
import jax.numpy as jnp
from jax import lax
from jax.experimental import pallas as pl
from jax.experimental.pallas import tpu as pltpu
from jax.experimental.pallas import tpu_sc as plsc

VOCAB = 100000
B = 32
T = 2048
D = 128
C = 128
NB = 16
NBUF = 6
LOOKAHEAD = 5
LANES = 16


def _emb_body(idx_hbm, wtr_hbm, wpe_hbm, out_hbm,
              idx_v, wpe_v, bufs, sems, sem_i, sem_p):
    h = lax.axis_index("c")
    tc = lax.axis_index("s")

    t0 = tc * C
    b0 = h * NB

    sem_g = sems[:NBUF]
    sem_w = sems[NBUF:]

    idx_cp = pltpu.async_copy(
        idx_hbm.at[pl.ds(b0, NB), pl.ds(t0, C)], idx_v, sem_i)
    wpe_cp = pltpu.async_copy(wpe_hbm.at[pl.ds(t0, C)], wpe_v, sem_p)
    idx_cp.wait()

    def start_gather(j):
        s = j % NBUF
        return pltpu.async_copy(wtr_hbm.at[idx_v.at[j]], bufs.at[s], sem_g[s])

    gd = [None] * NB
    wd = [None] * NB

    for j in range(LOOKAHEAD):
        gd[j] = start_gather(j)
    wpe_cp.wait()

    for j in range(NB):
        s = j % NBUF
        gd[j].wait()

        @pl.loop(0, C)
        def _per_row(r, s=s):
            for k in range(D // LANES):
                sl = pl.ds(k * LANES, LANES)
                plsc.addupdate(bufs.at[s, r, sl], wpe_v[r, sl])

        wd[j] = pltpu.async_copy(
            bufs.at[s], out_hbm.at[b0 + j, pl.ds(t0, C)], sem_w[s])

        nj = j + LOOKAHEAD
        if nj < NB:
            pj = nj - NBUF
            if pj >= 0:
                wd[pj].wait()
            gd[nj] = start_gather(nj)

    for j in range(NB - NBUF, NB):
        if wd[j] is not None and j >= 0:
            wd[j].wait()


@functools.partial(
    pl.kernel,
    out_type=jax.ShapeDtypeStruct((B, T, D), jnp.float32),
    mesh=plsc.VectorSubcoreMesh(core_axis_name="c", subcore_axis_name="s"),
    scratch_types=[
        pltpu.VMEM((NB, C), jnp.int32),
        pltpu.VMEM((C, D), jnp.float32),
        pltpu.VMEM((NBUF, C, D), jnp.float32),
        [pltpu.SemaphoreType.DMA] * (2 * NBUF),
        pltpu.SemaphoreType.DMA,
        pltpu.SemaphoreType.DMA,
    ],
)
def _emb_kernel(idx_hbm, wtr_hbm, wpe_hbm, out_hbm, idx_v, wpe_v, bufs, sems,
                sem_i, sem_p):
    _emb_body(idx_hbm, wtr_hbm, wpe_hbm, out_hbm, idx_v, wpe_v, bufs, sems,
              sem_i, sem_p)


def kernel(idx, wtr, wpe):
    return _emb_kernel(idx, wtr, wpe)
